# baseline (speedup 1.0000x reference)
import functools

import jax
import jax.numpy as jnp
from jax import lax
from jax.experimental import pallas as pl
from jax.experimental.pallas import tpu as pltpu
from jax.experimental.pallas import tpu_sc as plsc

VOCAB = 1_000_000
D = 64
NC = 2
NS = 16
NW = NC * NS
B = 4096 * 200
BPW = B // NW
ROW = 128
CHUNK_ROWS = 8
CHUNK = CHUNK_ROWS * ROW
NCHUNKS = BPW // CHUNK


def _embed_body(idx_hbm, w_hbm, out_hbm, idx_v, rows_v, sem):
    wid = lax.axis_index("s") * NC + lax.axis_index("c")

    def chunk(ci, _):
        base = pl.multiple_of(wid * BPW + ci * CHUNK, CHUNK)
        rbase = pl.multiple_of(wid * (BPW // ROW) + ci * CHUNK_ROWS, CHUNK_ROWS)
        pltpu.sync_copy(idx_hbm.at[pl.ds(rbase, CHUNK_ROWS)], idx_v)
        copies = [
            pltpu.async_copy(
                w_hbm.at[idx_v.at[j]], rows_v.at[pl.ds(j * ROW, ROW)], sem
            )
            for j in range(CHUNK_ROWS)
        ]
        for c in copies:
            c.wait()
        pltpu.sync_copy(rows_v, out_hbm.at[pl.ds(base, CHUNK)])
        return 0

    lax.fori_loop(0, NCHUNKS, chunk, 0)


_embed = functools.partial(
    pl.kernel,
    mesh=plsc.VectorSubcoreMesh(core_axis_name="c", subcore_axis_name="s"),
    out_type=jax.ShapeDtypeStruct((B, D), jnp.float32),
    scratch_types=[
        pltpu.VMEM((CHUNK_ROWS, ROW), jnp.int32),
        pltpu.VMEM((CHUNK, D), jnp.float32),
        pltpu.SemaphoreType.DMA,
    ],
    compiler_params=pltpu.CompilerParams(use_tc_tiling_on_sc=False),
)(_embed_body)


def kernel(idx, weight):
    idx2d = idx.reshape(B // ROW, ROW).astype(jnp.int32)
    out = _embed(idx2d, weight)
    return out.reshape(4096, 200, D)


# speedup vs baseline: 1.0174x; 1.0174x over previous
import functools

import jax
import jax.numpy as jnp
from jax import lax
from jax.experimental import pallas as pl
from jax.experimental.pallas import tpu as pltpu
from jax.experimental.pallas import tpu_sc as plsc

VOCAB = 1_000_000
D = 64
NC = 2
NS = 16
NW = NC * NS
B = 4096 * 200
BPW = B // NW
ROW = 128
NSTEPS = BPW // ROW
GROWS = 4
NG = NSTEPS // GROWS
NPAIR = NG // 2


def _embed_body(idx_hbm, w_hbm, out_hbm, idx_all, rows, gsem0, gsem1,
                wsem0, wsem1):
    wid = lax.axis_index("s") * NC + lax.axis_index("c")
    rbase = pl.multiple_of(wid * NSTEPS, 8)
    pltpu.sync_copy(idx_hbm.at[pl.ds(rbase, NSTEPS)], idx_all)
    obase = pl.multiple_of(wid * BPW, 8)
    gsem = (gsem0, gsem1)
    wsem = (wsem0, wsem1)

    def fire_g(g, p):
        for j in range(GROWS):
            pltpu.async_copy(w_hbm.at[idx_all.at[g * GROWS + j]],
                             rows.at[p * GROWS + j], gsem[p])

    def fire_w(g, p):
        for j in range(GROWS):
            pltpu.async_copy(
                rows.at[p * GROWS + j],
                out_hbm.at[pl.ds(obase + (g * GROWS + j) * ROW, ROW)],
                wsem[p])

    def drain_g(p):
        for _ in range(GROWS):
            pltpu.make_async_copy(w_hbm.at[pl.ds(0, ROW)], rows.at[0],
                                  gsem[p]).wait()

    def drain_w(p):
        for _ in range(GROWS):
            pltpu.make_async_copy(rows.at[0], out_hbm.at[pl.ds(obase, ROW)],
                                  wsem[p]).wait()

    fire_g(0, 0)

    def pair(k, _):
        g0 = 2 * k
        drain_g(0)

        @pl.when(k > 0)
        def _():
            drain_w(1)

        fire_g(g0 + 1, 1)
        fire_w(g0, 0)
        drain_g(1)
        drain_w(0)

        @pl.when(k < NPAIR - 1)
        def _():
            fire_g(g0 + 2, 0)

        fire_w(g0 + 1, 1)
        return 0

    lax.fori_loop(0, NPAIR, pair, 0)
    drain_w(1)


_embed = functools.partial(
    pl.kernel,
    mesh=plsc.VectorSubcoreMesh(core_axis_name="c", subcore_axis_name="s"),
    out_type=jax.ShapeDtypeStruct((B, D), jnp.float32),
    scratch_types=[
        pltpu.VMEM((NSTEPS, ROW), jnp.int32),
        pltpu.VMEM((2 * GROWS, ROW, D), jnp.float32),
        pltpu.SemaphoreType.DMA,
        pltpu.SemaphoreType.DMA,
        pltpu.SemaphoreType.DMA,
        pltpu.SemaphoreType.DMA,
    ],
    compiler_params=pltpu.CompilerParams(use_tc_tiling_on_sc=False),
)(_embed_body)


def kernel(idx, weight):
    idx2d = idx.reshape(B // ROW, ROW).astype(jnp.int32)
    out = _embed(idx2d, weight)
    return out.reshape(4096, 200, D)
